# Initial kernel scaffold; baseline (speedup 1.0000x reference)
import jax
import jax.numpy as jnp
from jax.experimental import pallas as pl


def kernel(cell, x, z, struct_size, emb, W_msg, b_msg, W_upd, b_upd, W1, b1, W2, b2):
    raise NotImplementedError("write your pallas kernel here")


# speedup vs baseline: 2.1725x; 2.1725x over previous
import jax
import jax.numpy as jnp
from jax.experimental import pallas as pl

_K = 16


def _silu(t):
    return t / (1.0 + jnp.exp(-t))


def _rne(t):
    return t.astype(jnp.bfloat16).astype(jnp.float32)


def _body(cell_r, x_r, xT_r, z_r, emb_r, wm_r, bm_r, wu_r, bu_r, w1_r,
          b1_r, w2_r, b2_r, out_r):
    f32 = jnp.float32
    n = x_r.shape[1]
    L = wm_r.shape[0]
    F = emb_r.shape[1]

    xs = x_r[0]
    frac = xs - jnp.floor(xs)
    xT = xT_r[0]
    fracT = xT - jnp.floor(xT)

    cellv = [[cell_r[0, i, j] for j in range(3)] for i in range(3)]
    d = []
    for a in range(3):
        t = frac[:, a:a + 1] - fracT[a:a + 1, :]
        t = t - jnp.round(t)
        d.append(t.astype(jnp.bfloat16).astype(f32))
    cart = [d[0] * cellv[0][c] + d[1] * cellv[1][c] + d[2] * cellv[2][c]
            for c in range(3)]
    rowi = jax.lax.broadcasted_iota(jnp.int32, (n, n), 0).astype(f32)
    colj = jax.lax.broadcasted_iota(jnp.int32, (n, n), 1).astype(f32)
    dist2 = cart[0] * cart[0] + cart[1] * cart[1] + cart[2] * cart[2]
    dist2 = dist2 + jnp.where(rowi == colj, 1e9, 0.0)

    D = dist2
    P_list, d_list = [], []
    u_lists = [[], [], []]
    for _ in range(_K):
        m = jnp.min(D, axis=1, keepdims=True)
        am = jnp.min(jnp.where(D == m, colj, float(n)), axis=1,
                     keepdims=True)
        Pk = (colj == am).astype(f32)
        dk = jnp.sqrt(jnp.maximum(m, 1e-12))
        P_list.append(Pk)
        d_list.append(dk)
        inv = 1.0 / (dk + 1e-9)
        for c in range(3):
            u_lists[c].append(
                jnp.sum(Pk * cart[c], axis=1, keepdims=True) * inv)
        D = jnp.where(Pk > 0.0, 1e9, D)
    P = jnp.concatenate(P_list, axis=0)
    dcol = jnp.concatenate(d_list, axis=0)
    U = [jnp.concatenate(u_lists[c], axis=0) for c in range(3)]

    zb = z_r[0]
    lane = jax.lax.broadcasted_iota(jnp.int32, (n, emb_r.shape[0]),
                                    1).astype(f32)
    oh = (lane == zb).astype(f32)
    h = jnp.dot(oh, emb_r[...], preferred_element_type=f32,
                precision=jax.lax.Precision.HIGHEST)

    dcol_r = _rne(dcol)

    def edge_mlp(hcur, Wi, Wj, Wd, bv):
        hr = _rne(hcur)
        hwi = jnp.dot(hr, _rne(Wi), preferred_element_type=f32)
        hwj = jnp.dot(hr, _rne(Wj), preferred_element_type=f32)
        hj = jnp.dot(P, hwj, preferred_element_type=f32,
                     precision=jax.lax.Precision.HIGHEST)
        hit = jnp.concatenate([hwi] * _K, axis=0)
        return _silu(hit + hj + dcol_r * _rne(Wd) + bv)

    for l in range(L):
        msg = edge_mlp(h, wm_r[l, 0:F, :], wm_r[l, F:2 * F, :],
                       wm_r[l, 2 * F:2 * F + 1, :], bm_r[l:l + 1, :])
        agg = msg[0:n, :]
        for kk in range(1, _K):
            agg = agg + msg[kk * n:(kk + 1) * n, :]
        upd = _silu(jnp.dot(_rne(h), _rne(wu_r[l, 0:F, :]),
                            preferred_element_type=f32)
                    + jnp.dot(_rne(agg), _rne(wu_r[l, F:2 * F, :]),
                              preferred_element_type=f32)
                    + bu_r[l:l + 1, :])
        h = h + upd

    u = edge_mlp(h, w1_r[0:F, :], w1_r[F:2 * F, :], w1_r[2 * F:2 * F + 1, :],
                 b1_r[...])
    w = (jnp.sum(_rne(u) * _rne(w2_r[...]), axis=1, keepdims=True)
         + b2_r[0, 0])
    disp = []
    for c in range(3):
        t = w * U[c]
        s = t[0:n, :]
        for kk in range(1, _K):
            s = s + t[kk * n:(kk + 1) * n, :]
        disp.append(s)
    out_r[0] = frac + jnp.concatenate(disp, axis=1)


def kernel(cell, x, z, struct_size, emb, W_msg, b_msg, W_upd, b_upd,
           W1, b1, W2, b2):
    del struct_size
    B = cell.shape[0]
    N = x.shape[0]
    n = N // B
    F = emb.shape[1]
    HID = W1.shape[1]
    x3 = x.reshape(B, n, 3)
    xT3 = jnp.swapaxes(x3, 1, 2)
    zf = z.astype(jnp.float32).reshape(B, n, 1)
    Vp = max(128, ((emb.shape[0] + 127) // 128) * 128)
    emb_p = jnp.zeros((Vp, F), jnp.float32).at[:emb.shape[0]].set(emb)
    b1r = b1.reshape(1, HID)
    w2r = W2.reshape(1, HID)
    b2r = b2.reshape(1, 1)

    out = pl.pallas_call(
        _body,
        grid=(B,),
        in_specs=[
            pl.BlockSpec((1, 3, 3), lambda b: (b, 0, 0)),
            pl.BlockSpec((1, n, 3), lambda b: (b, 0, 0)),
            pl.BlockSpec((1, 3, n), lambda b: (b, 0, 0)),
            pl.BlockSpec((1, n, 1), lambda b: (b, 0, 0)),
            pl.BlockSpec((Vp, F), lambda b: (0, 0)),
            pl.BlockSpec(W_msg.shape, lambda b: (0, 0, 0)),
            pl.BlockSpec(b_msg.shape, lambda b: (0, 0)),
            pl.BlockSpec(W_upd.shape, lambda b: (0, 0, 0)),
            pl.BlockSpec(b_upd.shape, lambda b: (0, 0)),
            pl.BlockSpec(W1.shape, lambda b: (0, 0)),
            pl.BlockSpec((1, HID), lambda b: (0, 0)),
            pl.BlockSpec((1, HID), lambda b: (0, 0)),
            pl.BlockSpec((1, 1), lambda b: (0, 0)),
        ],
        out_specs=pl.BlockSpec((1, n, 3), lambda b: (b, 0, 0)),
        out_shape=jax.ShapeDtypeStruct((B, n, 3), jnp.float32),
    )(cell, x3, xT3, zf, emb_p, W_msg, b_msg, W_upd, b_upd, W1, b1r, w2r, b2r)
    return out.reshape(N, 3)


# speedup vs baseline: 5.0607x; 2.3295x over previous
import jax
import jax.numpy as jnp
from jax.experimental import pallas as pl

_K = 16
_G = 8


def _silu(t):
    return t / (1.0 + jnp.exp(-t))


def _rne(t):
    return t.astype(jnp.bfloat16).astype(jnp.float32)


def _body(cellrep_r, x_r, xT_r, z_r, emb_r, wm_r, bm_r, wu_r, bu_r, w1_r,
          b1_r, w2_r, b2_r, out_r):
    f32 = jnp.float32
    G = x_r.shape[0]
    n = x_r.shape[1]
    Gn = G * n
    L = wm_r.shape[0]
    F = emb_r.shape[1]

    xs = x_r[...].reshape(Gn, 3)
    frac = xs - jnp.floor(xs)
    xT = xT_r[...]
    fT = xT - jnp.floor(xT)
    cr = _rne(cellrep_r[...].reshape(Gn, 9))

    d = []
    for a in range(3):
        fTa = jnp.broadcast_to(fT[:, a:a + 1, :], (G, n, n)).reshape(Gn, n)
        t = frac[:, a:a + 1] - fTa
        t = t - jnp.round(t)
        d.append(_rne(t))
    cart = [d[0] * cr[:, 0 + c:1 + c] + d[1] * cr[:, 3 + c:4 + c]
            + d[2] * cr[:, 6 + c:7 + c] for c in range(3)]

    rloc = jax.lax.broadcasted_iota(jnp.int32, (G, n, n), 1).reshape(Gn, n)
    cI = jax.lax.broadcasted_iota(jnp.int32, (Gn, n), 1)
    colj = cI.astype(f32)
    dist2 = cart[0] * cart[0] + cart[1] * cart[1] + cart[2] * cart[2]
    D = dist2 + jnp.where(rloc == cI, 1e9, 0.0)

    P_list, d_list = [], []
    u_lists = [[], [], []]
    for _ in range(_K):
        m = jnp.min(D, axis=1, keepdims=True)
        am = jnp.min(jnp.where(D == m, colj, float(n)), axis=1,
                     keepdims=True)
        Pk = (colj == am).astype(f32)
        dk = jnp.sqrt(jnp.maximum(m, 1e-12))
        P_list.append(Pk)
        d_list.append(dk)
        inv = 1.0 / (dk + 1e-9)
        for c in range(3):
            u_lists[c].append(
                jnp.sum(Pk * cart[c], axis=1, keepdims=True) * inv)
        D = D + Pk * 1e9
    dcol = jnp.concatenate(d_list, axis=0)
    U = [jnp.concatenate(u_lists[c], axis=0) for c in range(3)]
    dcol_r = _rne(dcol)

    P_gs = [jnp.concatenate([P_list[k][g * n:(g + 1) * n, :]
                             for k in range(_K)], axis=0)
            for g in range(G)]

    zb = z_r[...].reshape(Gn, 1)
    lane = jax.lax.broadcasted_iota(jnp.int32, (Gn, emb_r.shape[0]),
                                    1).astype(f32)
    oh = (lane == zb).astype(f32)
    h = jnp.dot(oh, emb_r[...], preferred_element_type=f32,
                precision=jax.lax.Precision.HIGHEST)

    def edge_mlp(hcur, Wi, Wj, Wd, bv):
        hr = _rne(hcur)
        hwi = jnp.dot(hr, _rne(Wi), preferred_element_type=f32)
        hwj = jnp.dot(hr, _rne(Wj), preferred_element_type=f32)
        hj_gs = [jnp.dot(P_gs[g], hwj[g * n:(g + 1) * n, :],
                         preferred_element_type=f32,
                         precision=jax.lax.Precision.HIGHEST)
                 for g in range(G)]
        hj = jnp.concatenate([hj_gs[g][k * n:(k + 1) * n, :]
                              for k in range(_K) for g in range(G)], axis=0)
        hit = jnp.concatenate([hwi] * _K, axis=0)
        return _silu(hit + hj + dcol_r * _rne(Wd) + bv)

    for l in range(L):
        msg = edge_mlp(h, wm_r[l, 0:F, :], wm_r[l, F:2 * F, :],
                       wm_r[l, 2 * F:2 * F + 1, :], bm_r[l:l + 1, :])
        agg = msg[0:Gn, :]
        for kk in range(1, _K):
            agg = agg + msg[kk * Gn:(kk + 1) * Gn, :]
        upd = _silu(jnp.dot(_rne(h), _rne(wu_r[l, 0:F, :]),
                            preferred_element_type=f32)
                    + jnp.dot(_rne(agg), _rne(wu_r[l, F:2 * F, :]),
                              preferred_element_type=f32)
                    + bu_r[l:l + 1, :])
        h = h + upd

    u = edge_mlp(h, w1_r[0:F, :], w1_r[F:2 * F, :], w1_r[2 * F:2 * F + 1, :],
                 b1_r[...])
    w = (jnp.sum(_rne(u) * _rne(w2_r[...]), axis=1, keepdims=True)
         + b2_r[0, 0])
    disp = []
    for c in range(3):
        t = w * U[c]
        s = t[0:Gn, :]
        for kk in range(1, _K):
            s = s + t[kk * Gn:(kk + 1) * Gn, :]
        disp.append(s)
    out = frac + jnp.concatenate(disp, axis=1)
    out_r[...] = out.reshape(G, n, 3)


def kernel(cell, x, z, struct_size, emb, W_msg, b_msg, W_upd, b_upd,
           W1, b1, W2, b2):
    del struct_size
    B = cell.shape[0]
    N = x.shape[0]
    n = N // B
    F = emb.shape[1]
    HID = W1.shape[1]
    G = _G
    x3 = x.reshape(B, n, 3)
    xT3 = jnp.swapaxes(x3, 1, 2)
    zf = z.astype(jnp.float32).reshape(B, n, 1)
    cellrep = jnp.broadcast_to(cell.reshape(B, 1, 9), (B, n, 9))
    Vp = max(128, ((emb.shape[0] + 127) // 128) * 128)
    emb_p = jnp.zeros((Vp, F), jnp.float32).at[:emb.shape[0]].set(emb)
    b1r = b1.reshape(1, HID)
    w2r = W2.reshape(1, HID)
    b2r = b2.reshape(1, 1)

    out = pl.pallas_call(
        _body,
        grid=(B // G,),
        in_specs=[
            pl.BlockSpec((G, n, 9), lambda b: (b, 0, 0)),
            pl.BlockSpec((G, n, 3), lambda b: (b, 0, 0)),
            pl.BlockSpec((G, 3, n), lambda b: (b, 0, 0)),
            pl.BlockSpec((G, n, 1), lambda b: (b, 0, 0)),
            pl.BlockSpec((Vp, F), lambda b: (0, 0)),
            pl.BlockSpec(W_msg.shape, lambda b: (0, 0, 0)),
            pl.BlockSpec(b_msg.shape, lambda b: (0, 0)),
            pl.BlockSpec(W_upd.shape, lambda b: (0, 0, 0)),
            pl.BlockSpec(b_upd.shape, lambda b: (0, 0)),
            pl.BlockSpec(W1.shape, lambda b: (0, 0)),
            pl.BlockSpec((1, HID), lambda b: (0, 0)),
            pl.BlockSpec((1, HID), lambda b: (0, 0)),
            pl.BlockSpec((1, 1), lambda b: (0, 0)),
        ],
        out_specs=pl.BlockSpec((G, n, 3), lambda b: (b, 0, 0)),
        out_shape=jax.ShapeDtypeStruct((B, n, 3), jnp.float32),
    )(cellrep, x3, xT3, zf, emb_p, W_msg, b_msg, W_upd, b_upd, W1, b1r,
      w2r, b2r)
    return out.reshape(N, 3)
